# baseline (speedup 1.0000x reference)
import functools

import jax
import jax.numpy as jnp
from jax import lax
from jax.experimental import pallas as pl
from jax.experimental.pallas import tpu as pltpu
from jax.experimental.pallas import tpu_sc as plsc

N = 10000
E = 320000
D = 128
DH = D // 2
NC, NS = 2, 16
CW = 125
ET = E // NS
NCHT = ET // CW
NCHH = NCHT // NC
NPAD = 10240
RT = NPAD // NS
BN_EPS = 1e-5

_mesh = plsc.VectorSubcoreMesh(core_axis_name="c", subcore_axis_name="s")


DW = 16


@functools.partial(
    pl.kernel,
    out_type=jax.ShapeDtypeStruct((NC, NPAD, DW), jnp.float32),
    mesh=_mesh,
    scratch_types=[
        pltpu.VMEM((NCHH, CW), jnp.int32),
        pltpu.VMEM((CW, DW), jnp.float32),
        pltpu.VMEM_SHARED((NPAD, DW), jnp.float32),
    ],
    compiler_params=pltpu.CompilerParams(use_tc_tiling_on_sc=False),
)
def _deg_kernel(row_hbm, ones_hbm, zeros_hbm, deg_out, row_v, ones_v, deg_sp):
    cid = lax.axis_index("c")
    sid = lax.axis_index("s")
    pltpu.sync_copy(row_hbm.at[sid, pl.ds(cid * NCHH, NCHH)], row_v)
    pltpu.sync_copy(ones_hbm, ones_v)
    pltpu.sync_copy(zeros_hbm, deg_sp.at[pl.ds(sid * RT, RT)])
    plsc.subcore_barrier()

    @pl.loop(0, NCHH)
    def _count(j):
        pltpu.sync_copy(ones_v, deg_sp.at[row_v.at[j]], add=True)

    plsc.subcore_barrier()
    pltpu.sync_copy(deg_sp.at[pl.ds(sid * RT, RT)],
                    deg_out.at[cid, pl.ds(sid * RT, RT)])


@functools.partial(
    pl.kernel,
    out_type=jax.ShapeDtypeStruct((NC, NPAD, DH), jnp.float32),
    mesh=_mesh,
    scratch_types=[
        pltpu.VMEM((NCHT, CW), jnp.int32),
        pltpu.VMEM((NCHT, CW), jnp.int32),
        pltpu.VMEM((CW, DH), jnp.float32),
        pltpu.VMEM((CW, DH), jnp.float32),
        pltpu.VMEM_SHARED((NPAD, DH), jnp.float32),
        pltpu.SemaphoreType.DMA,
        pltpu.SemaphoreType.DMA,
    ],
    compiler_params=pltpu.CompilerParams(use_tc_tiling_on_sc=False),
)
def _msg_kernel(yh_hbm, row_hbm, col_hbm, zeros_hbm, z_out,
                row_v, col_v, gbuf0, gbuf1, z_sp, gsem, ssem):
    cid = lax.axis_index("c")
    sid = lax.axis_index("s")
    y_src = yh_hbm.at[cid]
    pltpu.sync_copy(row_hbm.at[sid], row_v)
    pltpu.sync_copy(col_hbm.at[sid], col_v)
    pltpu.sync_copy(zeros_hbm, z_sp.at[pl.ds(sid * RT, RT)])
    plsc.subcore_barrier()

    bufs = (gbuf0, gbuf1)
    pltpu.async_copy(y_src.at[row_v.at[0]], gbuf0, gsem)

    @pl.loop(0, NCHT, step=2)
    def _chunks(j):
        for u in range(2):
            jj = j + u
            buf = bufs[u]
            nbuf = bufs[(u + 1) % 2]
            pltpu.make_async_copy(y_src.at[row_v.at[jj]], buf, gsem).wait()
            pltpu.async_copy(buf, z_sp.at[col_v.at[jj]], ssem, add=True)

            @pl.when(jj >= 1)
            def _drain_prev():
                pltpu.make_async_copy(
                    nbuf, z_sp.at[col_v.at[jj - 1]], ssem).wait()

            @pl.when(jj + 1 < NCHT)
            def _prefetch():
                pltpu.async_copy(y_src.at[row_v.at[jj + 1]], nbuf, gsem)

    pltpu.make_async_copy(gbuf1, z_sp.at[col_v.at[NCHT - 1]], ssem).wait()
    plsc.subcore_barrier()
    pltpu.sync_copy(z_sp.at[pl.ds(sid * RT, RT)],
                    z_out.at[cid, pl.ds(sid * RT, RT)])


_TB = 1000


def _lin_body(x_ref, w_ref, b_ref, degs_ref, yh_ref, dis_ref):
    xw = lax.dot_general(x_ref[...], w_ref[...], (((1,), (1,)), ((), ())),
                         preferred_element_type=jnp.float32)
    xw = xw + b_ref[...]
    deg = degs_ref[0, :, 0] + degs_ref[1, :, 0] + 1.0
    dis = lax.rsqrt(deg)[:, None]
    y = dis * xw
    yh_ref[0] = y[:, :DH]
    yh_ref[1] = y[:, DH:]
    dis_ref[...] = dis


def _lin_call(x, W, b2, deg_parts):
    return pl.pallas_call(
        _lin_body,
        grid=(N // _TB,),
        in_specs=[
            pl.BlockSpec((_TB, D), lambda i: (i, 0)),
            pl.BlockSpec((D, D), lambda i: (0, 0)),
            pl.BlockSpec((1, D), lambda i: (0, 0)),
            pl.BlockSpec((NC, _TB, DW), lambda i: (0, i, 0)),
        ],
        out_specs=[
            pl.BlockSpec((NC, _TB, DH), lambda i: (0, i, 0)),
            pl.BlockSpec((_TB, 1), lambda i: (i, 0)),
        ],
        out_shape=[
            jax.ShapeDtypeStruct((NC, N, DH), jnp.float32),
            jax.ShapeDtypeStruct((N, 1), jnp.float32),
        ],
    )(x, W, b2, deg_parts)


def _bn_body(zp_ref, yh_ref, dis_ref, g_ref, be_ref, o_ref):
    tl = zp_ref[0, :N, :] + yh_ref[0]
    tr = zp_ref[1, :N, :] + yh_ref[1]
    t = jnp.concatenate([tl, tr], axis=1) * dis_ref[...]
    m = jnp.mean(t, axis=0, keepdims=True)
    d = t - m
    v = jnp.mean(d * d, axis=0, keepdims=True)
    o_ref[...] = jnp.maximum(
        d * lax.rsqrt(v + BN_EPS) * g_ref[...] + be_ref[...], 0.0)


def _bn_call(z_parts, yh, dis, g2, be2):
    return pl.pallas_call(
        _bn_body,
        out_shape=jax.ShapeDtypeStruct((N, D), jnp.float32),
    )(z_parts, yh, dis, g2, be2)


def kernel(x, edge_index, W, b, bn_gamma, bn_beta):
    ei = edge_index.astype(jnp.int32)
    row = ei[0].reshape(NS, NCHT, CW)
    col = ei[1].reshape(NS, NCHT, CW)
    zeros = jnp.zeros((RT, DH), jnp.float32)
    ones8 = jnp.ones((CW, DW), jnp.float32)
    zeros8 = jnp.zeros((RT, DW), jnp.float32)

    deg_parts = _deg_kernel(row, ones8, zeros8)
    yh, dis = _lin_call(x, W, b.reshape(1, D), deg_parts)
    z_parts = _msg_kernel(yh, row, col, zeros)
    out = _bn_call(z_parts, yh, dis,
                   bn_gamma.reshape(1, D), bn_beta.reshape(1, D))
    return out


# speedup vs baseline: 1.2151x; 1.2151x over previous
import functools

import jax
import jax.numpy as jnp
from jax import lax
from jax.experimental import pallas as pl
from jax.experimental.pallas import tpu as pltpu
from jax.experimental.pallas import tpu_sc as plsc

N = 10000
E = 320000
D = 128
DH = D // 2
NC, NS = 2, 16
CW = 250
ET = E // NS
NCHT = ET // CW
NCHH = NCHT // NC
NPAD = 10240
RT = NPAD // NS
BN_EPS = 1e-5

_mesh = plsc.VectorSubcoreMesh(core_axis_name="c", subcore_axis_name="s")


DW = 16


@functools.partial(
    pl.kernel,
    out_type=jax.ShapeDtypeStruct((NC, NPAD, DW), jnp.float32),
    mesh=_mesh,
    scratch_types=[
        pltpu.VMEM((NCHH, CW), jnp.int32),
        pltpu.VMEM((CW, DW), jnp.float32),
        pltpu.VMEM_SHARED((NPAD, DW), jnp.float32),
    ],
    compiler_params=pltpu.CompilerParams(use_tc_tiling_on_sc=False),
)
def _deg_kernel(row_hbm, ones_hbm, zeros_hbm, deg_out, row_v, ones_v, deg_sp):
    cid = lax.axis_index("c")
    sid = lax.axis_index("s")
    pltpu.sync_copy(row_hbm.at[sid, pl.ds(cid * NCHH, NCHH)], row_v)
    pltpu.sync_copy(ones_hbm, ones_v)
    pltpu.sync_copy(zeros_hbm, deg_sp.at[pl.ds(sid * RT, RT)])
    plsc.subcore_barrier()

    @pl.loop(0, NCHH)
    def _count(j):
        pltpu.sync_copy(ones_v, deg_sp.at[row_v.at[j]], add=True)

    plsc.subcore_barrier()
    pltpu.sync_copy(deg_sp.at[pl.ds(sid * RT, RT)],
                    deg_out.at[cid, pl.ds(sid * RT, RT)])


@functools.partial(
    pl.kernel,
    out_type=jax.ShapeDtypeStruct((NC, NPAD, DH), jnp.float32),
    mesh=_mesh,
    scratch_types=[
        pltpu.VMEM((NCHT, CW), jnp.int32),
        pltpu.VMEM((NCHT, CW), jnp.int32),
        pltpu.VMEM((CW, DH), jnp.float32),
        pltpu.VMEM((CW, DH), jnp.float32),
        pltpu.VMEM_SHARED((NPAD, DH), jnp.float32),
        pltpu.SemaphoreType.DMA,
        pltpu.SemaphoreType.DMA,
    ],
    compiler_params=pltpu.CompilerParams(use_tc_tiling_on_sc=False),
)
def _msg_kernel(yh_hbm, row_hbm, col_hbm, zeros_hbm, z_out,
                row_v, col_v, gbuf0, gbuf1, z_sp, gsem, ssem):
    cid = lax.axis_index("c")
    sid = lax.axis_index("s")
    y_src = yh_hbm.at[cid]
    pltpu.sync_copy(row_hbm.at[sid], row_v)
    pltpu.sync_copy(col_hbm.at[sid], col_v)
    pltpu.sync_copy(zeros_hbm, z_sp.at[pl.ds(sid * RT, RT)])
    plsc.subcore_barrier()

    bufs = (gbuf0, gbuf1)
    pltpu.async_copy(y_src.at[row_v.at[0]], gbuf0, gsem)

    @pl.loop(0, NCHT, step=2)
    def _chunks(j):
        for u in range(2):
            jj = j + u
            buf = bufs[u]
            nbuf = bufs[(u + 1) % 2]
            pltpu.make_async_copy(y_src.at[row_v.at[jj]], buf, gsem).wait()
            pltpu.async_copy(buf, z_sp.at[col_v.at[jj]], ssem, add=True)

            @pl.when(jj >= 1)
            def _drain_prev():
                pltpu.make_async_copy(
                    nbuf, z_sp.at[col_v.at[jj - 1]], ssem).wait()

            @pl.when(jj + 1 < NCHT)
            def _prefetch():
                pltpu.async_copy(y_src.at[row_v.at[jj + 1]], nbuf, gsem)

    pltpu.make_async_copy(gbuf1, z_sp.at[col_v.at[NCHT - 1]], ssem).wait()
    plsc.subcore_barrier()
    pltpu.sync_copy(z_sp.at[pl.ds(sid * RT, RT)],
                    z_out.at[cid, pl.ds(sid * RT, RT)])


_TB = 1000


def _lin_body(x_ref, w_ref, b_ref, degs_ref, yh_ref, dis_ref):
    xw = lax.dot_general(x_ref[...], w_ref[...], (((1,), (1,)), ((), ())),
                         preferred_element_type=jnp.float32)
    xw = xw + b_ref[...]
    deg = degs_ref[0, :, 0] + degs_ref[1, :, 0] + 1.0
    dis = lax.rsqrt(deg)[:, None]
    y = dis * xw
    yh_ref[0] = y[:, :DH]
    yh_ref[1] = y[:, DH:]
    dis_ref[...] = dis


def _lin_call(x, W, b2, deg_parts):
    return pl.pallas_call(
        _lin_body,
        grid=(N // _TB,),
        in_specs=[
            pl.BlockSpec((_TB, D), lambda i: (i, 0)),
            pl.BlockSpec((D, D), lambda i: (0, 0)),
            pl.BlockSpec((1, D), lambda i: (0, 0)),
            pl.BlockSpec((NC, _TB, DW), lambda i: (0, i, 0)),
        ],
        out_specs=[
            pl.BlockSpec((NC, _TB, DH), lambda i: (0, i, 0)),
            pl.BlockSpec((_TB, 1), lambda i: (i, 0)),
        ],
        out_shape=[
            jax.ShapeDtypeStruct((NC, N, DH), jnp.float32),
            jax.ShapeDtypeStruct((N, 1), jnp.float32),
        ],
    )(x, W, b2, deg_parts)


def _bn_body(zp_ref, yh_ref, dis_ref, g_ref, be_ref, o_ref):
    tl = zp_ref[0, :N, :] + yh_ref[0]
    tr = zp_ref[1, :N, :] + yh_ref[1]
    t = jnp.concatenate([tl, tr], axis=1) * dis_ref[...]
    m = jnp.mean(t, axis=0, keepdims=True)
    d = t - m
    v = jnp.mean(d * d, axis=0, keepdims=True)
    o_ref[...] = jnp.maximum(
        d * lax.rsqrt(v + BN_EPS) * g_ref[...] + be_ref[...], 0.0)


def _bn_call(z_parts, yh, dis, g2, be2):
    return pl.pallas_call(
        _bn_body,
        out_shape=jax.ShapeDtypeStruct((N, D), jnp.float32),
    )(z_parts, yh, dis, g2, be2)


def kernel(x, edge_index, W, b, bn_gamma, bn_beta):
    ei = edge_index.astype(jnp.int32)
    row = ei[0].reshape(NS, NCHT, CW)
    col = ei[1].reshape(NS, NCHT, CW)
    zeros = jnp.zeros((RT, DH), jnp.float32)
    ones8 = jnp.ones((CW, DW), jnp.float32)
    zeros8 = jnp.zeros((RT, DW), jnp.float32)

    deg_parts = _deg_kernel(row, ones8, zeros8)
    yh, dis = _lin_call(x, W, b.reshape(1, D), deg_parts)
    z_parts = _msg_kernel(yh, row, col, zeros)
    out = _bn_call(z_parts, yh, dis,
                   bn_gamma.reshape(1, D), bn_beta.reshape(1, D))
    return out


# speedup vs baseline: 1.2905x; 1.0620x over previous
import functools

import jax
import jax.numpy as jnp
from jax import lax
from jax.experimental import pallas as pl
from jax.experimental.pallas import tpu as pltpu
from jax.experimental.pallas import tpu_sc as plsc

N = 10000
E = 320000
D = 128
DH = D // 2
NC, NS = 2, 16
CW = 500
ET = E // NS
NCHT = ET // CW
NCHH = NCHT // NC
NPAD = 10240
RT = NPAD // NS
BN_EPS = 1e-5

_mesh = plsc.VectorSubcoreMesh(core_axis_name="c", subcore_axis_name="s")


DW = 16


@functools.partial(
    pl.kernel,
    out_type=jax.ShapeDtypeStruct((NC, NPAD, DW), jnp.float32),
    mesh=_mesh,
    scratch_types=[
        pltpu.VMEM((NCHH, CW), jnp.int32),
        pltpu.VMEM((CW, DW), jnp.float32),
        pltpu.VMEM_SHARED((NPAD, DW), jnp.float32),
    ],
    compiler_params=pltpu.CompilerParams(use_tc_tiling_on_sc=False),
)
def _deg_kernel(row_hbm, ones_hbm, zeros_hbm, deg_out, row_v, ones_v, deg_sp):
    cid = lax.axis_index("c")
    sid = lax.axis_index("s")
    pltpu.sync_copy(row_hbm.at[sid, pl.ds(cid * NCHH, NCHH)], row_v)
    pltpu.sync_copy(ones_hbm, ones_v)
    pltpu.sync_copy(zeros_hbm, deg_sp.at[pl.ds(sid * RT, RT)])
    plsc.subcore_barrier()

    @pl.loop(0, NCHH)
    def _count(j):
        pltpu.sync_copy(ones_v, deg_sp.at[row_v.at[j]], add=True)

    plsc.subcore_barrier()
    pltpu.sync_copy(deg_sp.at[pl.ds(sid * RT, RT)],
                    deg_out.at[cid, pl.ds(sid * RT, RT)])


assert NCHT % 4 == 0


@functools.partial(
    pl.kernel,
    out_type=jax.ShapeDtypeStruct((NC, NPAD, DH), jnp.float32),
    mesh=_mesh,
    scratch_types=[
        pltpu.VMEM((4, CW), jnp.int32),
        pltpu.VMEM((4, CW), jnp.int32),
        pltpu.VMEM((CW, DH), jnp.float32),
        pltpu.VMEM((CW, DH), jnp.float32),
        pltpu.VMEM_SHARED((NPAD, DH), jnp.float32),
        pltpu.SemaphoreType.DMA,
        pltpu.SemaphoreType.DMA,
        pltpu.SemaphoreType.DMA,
    ],
    compiler_params=pltpu.CompilerParams(use_tc_tiling_on_sc=False),
)
def _msg_kernel(yh_hbm, row_hbm, col_hbm, zeros_hbm, z_out,
                row_v, col_v, gbuf0, gbuf1, z_sp, gsem, ssem, isem):
    cid = lax.axis_index("c")
    sid = lax.axis_index("s")
    y_src = yh_hbm.at[cid]
    pltpu.sync_copy(row_hbm.at[sid, 0], row_v.at[0])
    pltpu.sync_copy(col_hbm.at[sid, 0], col_v.at[0])
    for k in (1, 2):
        pltpu.async_copy(row_hbm.at[sid, k], row_v.at[k], isem)
        pltpu.async_copy(col_hbm.at[sid, k], col_v.at[k], isem)
    pltpu.sync_copy(zeros_hbm, z_sp.at[pl.ds(sid * RT, RT)])
    plsc.subcore_barrier()

    bufs = (gbuf0, gbuf1)
    pltpu.async_copy(y_src.at[row_v.at[0]], gbuf0, gsem)

    @pl.loop(0, NCHT, step=4)
    def _chunks(j):
        for u in range(4):
            jj = j + u
            buf = bufs[u % 2]
            nbuf = bufs[(u + 1) % 2]
            i_cur, i_nxt, i_old = u, (u + 1) % 4, (u + 3) % 4
            pltpu.make_async_copy(y_src.at[row_v.at[i_cur]], buf, gsem).wait()
            pltpu.async_copy(buf, z_sp.at[col_v.at[i_cur]], ssem, add=True)

            @pl.when(jj >= 1)
            def _drain_prev():
                pltpu.make_async_copy(
                    nbuf, z_sp.at[col_v.at[i_old]], ssem).wait()

            @pl.when(jj + 1 < NCHT)
            def _prefetch():
                pltpu.make_async_copy(
                    row_hbm.at[sid, jj + 1], row_v.at[i_nxt], isem).wait()
                pltpu.make_async_copy(
                    col_hbm.at[sid, jj + 1], col_v.at[i_nxt], isem).wait()
                pltpu.async_copy(y_src.at[row_v.at[i_nxt]], nbuf, gsem)

            @pl.when(jj + 3 < NCHT)
            def _stage_idx():
                pltpu.async_copy(row_hbm.at[sid, jj + 3], row_v.at[i_old],
                                 isem)
                pltpu.async_copy(col_hbm.at[sid, jj + 3], col_v.at[i_old],
                                 isem)

    pltpu.make_async_copy(bufs[(NCHT - 1) % 2],
                          z_sp.at[col_v.at[3]], ssem).wait()
    plsc.subcore_barrier()
    pltpu.sync_copy(z_sp.at[pl.ds(sid * RT, RT)],
                    z_out.at[cid, pl.ds(sid * RT, RT)])


_TB = 1000


def _lin_body(x_ref, w_ref, b_ref, degs_ref, yh_ref, dis_ref):
    xw = lax.dot_general(x_ref[...], w_ref[...], (((1,), (1,)), ((), ())),
                         preferred_element_type=jnp.float32)
    xw = xw + b_ref[...]
    deg = degs_ref[0, :, 0] + degs_ref[1, :, 0] + 1.0
    dis = lax.rsqrt(deg)[:, None]
    y = dis * xw
    yh_ref[0] = y[:, :DH]
    yh_ref[1] = y[:, DH:]
    dis_ref[...] = dis


def _lin_call(x, W, b2, deg_parts):
    return pl.pallas_call(
        _lin_body,
        grid=(N // _TB,),
        in_specs=[
            pl.BlockSpec((_TB, D), lambda i: (i, 0)),
            pl.BlockSpec((D, D), lambda i: (0, 0)),
            pl.BlockSpec((1, D), lambda i: (0, 0)),
            pl.BlockSpec((NC, _TB, DW), lambda i: (0, i, 0)),
        ],
        out_specs=[
            pl.BlockSpec((NC, _TB, DH), lambda i: (0, i, 0)),
            pl.BlockSpec((_TB, 1), lambda i: (i, 0)),
        ],
        out_shape=[
            jax.ShapeDtypeStruct((NC, N, DH), jnp.float32),
            jax.ShapeDtypeStruct((N, 1), jnp.float32),
        ],
    )(x, W, b2, deg_parts)


def _bn_body(zp_ref, yh_ref, dis_ref, g_ref, be_ref, o_ref):
    tl = zp_ref[0, :N, :] + yh_ref[0]
    tr = zp_ref[1, :N, :] + yh_ref[1]
    t = jnp.concatenate([tl, tr], axis=1) * dis_ref[...]
    m = jnp.mean(t, axis=0, keepdims=True)
    d = t - m
    v = jnp.mean(d * d, axis=0, keepdims=True)
    o_ref[...] = jnp.maximum(
        d * lax.rsqrt(v + BN_EPS) * g_ref[...] + be_ref[...], 0.0)


def _bn_call(z_parts, yh, dis, g2, be2):
    return pl.pallas_call(
        _bn_body,
        out_shape=jax.ShapeDtypeStruct((N, D), jnp.float32),
    )(z_parts, yh, dis, g2, be2)


def kernel(x, edge_index, W, b, bn_gamma, bn_beta):
    ei = edge_index.astype(jnp.int32)
    row = ei[0].reshape(NS, NCHT, CW)
    col = ei[1].reshape(NS, NCHT, CW)
    zeros = jnp.zeros((RT, DH), jnp.float32)
    ones8 = jnp.ones((CW, DW), jnp.float32)
    zeros8 = jnp.zeros((RT, DW), jnp.float32)

    deg_parts = _deg_kernel(row, ones8, zeros8)
    yh, dis = _lin_call(x, W, b.reshape(1, D), deg_parts)
    z_parts = _msg_kernel(yh, row, col, zeros)
    out = _bn_call(z_parts, yh, dis,
                   bn_gamma.reshape(1, D), bn_beta.reshape(1, D))
    return out


# speedup vs baseline: 1.3043x; 1.0107x over previous
import functools

import jax
import jax.numpy as jnp
from jax import lax
from jax.experimental import pallas as pl
from jax.experimental.pallas import tpu as pltpu
from jax.experimental.pallas import tpu_sc as plsc

N = 10000
E = 320000
D = 128
DH = D // 2
NC, NS = 2, 16
CW = 625
ET = E // NS
NCHT = ET // CW
NCHH = NCHT // NC
NPAD = 10240
RT = NPAD // NS
BN_EPS = 1e-5

_mesh = plsc.VectorSubcoreMesh(core_axis_name="c", subcore_axis_name="s")


DW = 16


@functools.partial(
    pl.kernel,
    out_type=jax.ShapeDtypeStruct((NC, NPAD, 8), jnp.float32),
    mesh=_mesh,
    scratch_types=[
        pltpu.VMEM((NCHH, CW), jnp.int32),
        pltpu.VMEM((CW, DW), jnp.float32),
        pltpu.VMEM_SHARED((NPAD, DW), jnp.float32),
    ],
    compiler_params=pltpu.CompilerParams(use_tc_tiling_on_sc=False),
)
def _deg_kernel(ei_hbm, ones_hbm, zeros_hbm, deg_out, row_v, ones_v, deg_sp):
    cid = lax.axis_index("c")
    sid = lax.axis_index("s")
    pltpu.sync_copy(ei_hbm.at[0, sid, pl.ds(cid * NCHH, NCHH)], row_v)
    pltpu.sync_copy(ones_hbm, ones_v)
    pltpu.sync_copy(zeros_hbm, deg_sp.at[pl.ds(sid * RT, RT)])
    plsc.subcore_barrier()

    @pl.loop(0, NCHH)
    def _count(j):
        pltpu.sync_copy(ones_v, deg_sp.at[row_v.at[j]], add=True)

    plsc.subcore_barrier()
    pltpu.sync_copy(deg_sp.at[pl.ds(sid * RT, RT), pl.ds(0, 8)],
                    deg_out.at[cid, pl.ds(sid * RT, RT)])


assert NCHT % 4 == 0


@functools.partial(
    pl.kernel,
    out_type=jax.ShapeDtypeStruct((NC, NPAD, DH), jnp.float32),
    mesh=_mesh,
    scratch_types=[
        pltpu.VMEM((4, CW), jnp.int32),
        pltpu.VMEM((4, CW), jnp.int32),
        pltpu.VMEM((CW, DH), jnp.float32),
        pltpu.VMEM((CW, DH), jnp.float32),
        pltpu.VMEM_SHARED((NPAD, DH), jnp.float32),
        pltpu.SemaphoreType.DMA,
        pltpu.SemaphoreType.DMA,
        pltpu.SemaphoreType.DMA,
    ],
    compiler_params=pltpu.CompilerParams(use_tc_tiling_on_sc=False),
)
def _msg_kernel(yh_hbm, ei_hbm, zeros_hbm, z_out,
                row_v, col_v, gbuf0, gbuf1, z_sp, gsem, ssem, isem):
    cid = lax.axis_index("c")
    sid = lax.axis_index("s")
    y_src = yh_hbm.at[cid]
    pltpu.sync_copy(ei_hbm.at[0, sid, 0], row_v.at[0])
    pltpu.sync_copy(ei_hbm.at[1, sid, 0], col_v.at[0])
    for k in (1, 2):
        pltpu.async_copy(ei_hbm.at[0, sid, k], row_v.at[k], isem)
        pltpu.async_copy(ei_hbm.at[1, sid, k], col_v.at[k], isem)
    pltpu.sync_copy(zeros_hbm, z_sp.at[pl.ds(sid * RT, RT)])
    plsc.subcore_barrier()

    bufs = (gbuf0, gbuf1)
    pltpu.async_copy(y_src.at[row_v.at[0]], gbuf0, gsem)

    @pl.loop(0, NCHT, step=4)
    def _chunks(j):
        for u in range(4):
            jj = j + u
            buf = bufs[u % 2]
            nbuf = bufs[(u + 1) % 2]
            i_cur, i_nxt, i_old = u, (u + 1) % 4, (u + 3) % 4
            pltpu.make_async_copy(y_src.at[row_v.at[i_cur]], buf, gsem).wait()
            pltpu.async_copy(buf, z_sp.at[col_v.at[i_cur]], ssem, add=True)

            @pl.when(jj >= 1)
            def _drain_prev():
                pltpu.make_async_copy(
                    nbuf, z_sp.at[col_v.at[i_old]], ssem).wait()

            @pl.when(jj + 1 < NCHT)
            def _prefetch():
                pltpu.make_async_copy(
                    ei_hbm.at[0, sid, jj + 1], row_v.at[i_nxt], isem).wait()
                pltpu.make_async_copy(
                    ei_hbm.at[1, sid, jj + 1], col_v.at[i_nxt], isem).wait()
                pltpu.async_copy(y_src.at[row_v.at[i_nxt]], nbuf, gsem)

            @pl.when(jj + 3 < NCHT)
            def _stage_idx():
                pltpu.async_copy(ei_hbm.at[0, sid, jj + 3], row_v.at[i_old],
                                 isem)
                pltpu.async_copy(ei_hbm.at[1, sid, jj + 3], col_v.at[i_old],
                                 isem)

    pltpu.make_async_copy(bufs[(NCHT - 1) % 2],
                          z_sp.at[col_v.at[3]], ssem).wait()
    plsc.subcore_barrier()
    pltpu.sync_copy(z_sp.at[pl.ds(sid * RT, RT)],
                    z_out.at[cid, pl.ds(sid * RT, RT)])


_TB = 1000


def _lin_body(x_ref, w_ref, b_ref, degs_ref, yh_ref, dis_ref):
    xw = lax.dot_general(x_ref[...], w_ref[...], (((1,), (1,)), ((), ())),
                         preferred_element_type=jnp.float32)
    xw = xw + b_ref[...]
    deg = degs_ref[0, :, 0] + degs_ref[1, :, 0] + 1.0
    dis = lax.rsqrt(deg)[:, None]
    y = dis * xw
    yh_ref[0] = y[:, :DH]
    yh_ref[1] = y[:, DH:]
    dis_ref[...] = dis


def _lin_call(x, W, b2, deg_parts):
    return pl.pallas_call(
        _lin_body,
        grid=(N // _TB,),
        in_specs=[
            pl.BlockSpec((_TB, D), lambda i: (i, 0)),
            pl.BlockSpec((D, D), lambda i: (0, 0)),
            pl.BlockSpec((1, D), lambda i: (0, 0)),
            pl.BlockSpec((NC, _TB, 8), lambda i: (0, i, 0)),
        ],
        out_specs=[
            pl.BlockSpec((NC, _TB, DH), lambda i: (0, i, 0)),
            pl.BlockSpec((_TB, 1), lambda i: (i, 0)),
        ],
        out_shape=[
            jax.ShapeDtypeStruct((NC, N, DH), jnp.float32),
            jax.ShapeDtypeStruct((N, 1), jnp.float32),
        ],
    )(x, W, b2, deg_parts)


def _bn_body(zp_ref, yh_ref, dis_ref, g_ref, be_ref, o_ref):
    tl = zp_ref[0, :N, :] + yh_ref[0]
    tr = zp_ref[1, :N, :] + yh_ref[1]
    t = jnp.concatenate([tl, tr], axis=1) * dis_ref[...]
    m = jnp.mean(t, axis=0, keepdims=True)
    d = t - m
    v = jnp.mean(d * d, axis=0, keepdims=True)
    o_ref[...] = jnp.maximum(
        d * lax.rsqrt(v + BN_EPS) * g_ref[...] + be_ref[...], 0.0)


def _bn_call(z_parts, yh, dis, g2, be2):
    return pl.pallas_call(
        _bn_body,
        out_shape=jax.ShapeDtypeStruct((N, D), jnp.float32),
    )(z_parts, yh, dis, g2, be2)


def kernel(x, edge_index, W, b, bn_gamma, bn_beta):
    eic = edge_index.astype(jnp.int32).reshape(2, NS, NCHT, CW)
    zeros = jnp.zeros((RT, DH), jnp.float32)
    ones8 = jnp.ones((CW, DW), jnp.float32)
    zeros8 = jnp.zeros((RT, DW), jnp.float32)

    deg_parts = _deg_kernel(eic, ones8, zeros8)
    yh, dis = _lin_call(x, W, b.reshape(1, D), deg_parts)
    z_parts = _msg_kernel(yh, eic, zeros)
    out = _bn_call(z_parts, yh, dis,
                   bn_gamma.reshape(1, D), bn_beta.reshape(1, D))
    return out


# speedup vs baseline: 1.3186x; 1.0110x over previous
import functools

import jax
import jax.numpy as jnp
from jax import lax
from jax.experimental import pallas as pl
from jax.experimental.pallas import tpu as pltpu
from jax.experimental.pallas import tpu_sc as plsc

N = 10000
E = 320000
D = 128
DH = D // 2
NC, NS = 2, 16
CW = 625
ET = E // NS
NCHT = ET // CW
NCHH = NCHT // NC
NPAD = 10240
RT = NPAD // NS
BN_EPS = 1e-5

_mesh = plsc.VectorSubcoreMesh(core_axis_name="c", subcore_axis_name="s")


DW = 16


@functools.partial(
    pl.kernel,
    out_type=jax.ShapeDtypeStruct((NC, NPAD, 8), jnp.float32),
    mesh=_mesh,
    scratch_types=[
        pltpu.VMEM((NCHH, CW), jnp.int32),
        pltpu.VMEM((CW, DW), jnp.float32),
        pltpu.VMEM_SHARED((NPAD, DW), jnp.float32),
        pltpu.SemaphoreType.DMA,
    ],
    compiler_params=pltpu.CompilerParams(use_tc_tiling_on_sc=False),
)
def _deg_kernel(ei_hbm, ones_hbm, zeros_hbm, deg_out, row_v, ones_v, deg_sp,
                ssem):
    cid = lax.axis_index("c")
    sid = lax.axis_index("s")
    pltpu.sync_copy(ei_hbm.at[0, sid, pl.ds(cid * NCHH, NCHH)], row_v)
    pltpu.sync_copy(ones_hbm, ones_v)
    pltpu.sync_copy(zeros_hbm, deg_sp.at[pl.ds(sid * RT, RT)])
    plsc.subcore_barrier()

    @pl.loop(0, NCHH)
    def _fire(j):
        pltpu.async_copy(ones_v, deg_sp.at[row_v.at[j]], ssem, add=True)

    @pl.loop(0, NCHH)
    def _drain(j):
        pltpu.make_async_copy(ones_v, deg_sp.at[row_v.at[j]], ssem).wait()

    plsc.subcore_barrier()
    pltpu.sync_copy(deg_sp.at[pl.ds(sid * RT, RT), pl.ds(0, 8)],
                    deg_out.at[cid, pl.ds(sid * RT, RT)])


assert NCHT % 4 == 0


@functools.partial(
    pl.kernel,
    out_type=jax.ShapeDtypeStruct((NC, NPAD, DH), jnp.float32),
    mesh=_mesh,
    scratch_types=[
        pltpu.VMEM((4, CW), jnp.int32),
        pltpu.VMEM((4, CW), jnp.int32),
        pltpu.VMEM((CW, DH), jnp.float32),
        pltpu.VMEM((CW, DH), jnp.float32),
        pltpu.VMEM_SHARED((NPAD, DH), jnp.float32),
        pltpu.SemaphoreType.DMA,
        pltpu.SemaphoreType.DMA,
        pltpu.SemaphoreType.DMA,
    ],
    compiler_params=pltpu.CompilerParams(use_tc_tiling_on_sc=False),
)
def _msg_kernel(yh_hbm, ei_hbm, zeros_hbm, z_out,
                row_v, col_v, gbuf0, gbuf1, z_sp, gsem, ssem, isem):
    cid = lax.axis_index("c")
    sid = lax.axis_index("s")
    y_src = yh_hbm.at[cid]
    pltpu.sync_copy(ei_hbm.at[0, sid, 0], row_v.at[0])
    pltpu.sync_copy(ei_hbm.at[1, sid, 0], col_v.at[0])
    for k in (1, 2):
        pltpu.async_copy(ei_hbm.at[0, sid, k], row_v.at[k], isem)
        pltpu.async_copy(ei_hbm.at[1, sid, k], col_v.at[k], isem)
    pltpu.sync_copy(zeros_hbm, z_sp.at[pl.ds(sid * RT, RT)])
    plsc.subcore_barrier()

    bufs = (gbuf0, gbuf1)
    pltpu.async_copy(y_src.at[row_v.at[0]], gbuf0, gsem)

    @pl.loop(0, NCHT, step=4)
    def _chunks(j):
        for u in range(4):
            jj = j + u
            buf = bufs[u % 2]
            nbuf = bufs[(u + 1) % 2]
            i_cur, i_nxt, i_old = u, (u + 1) % 4, (u + 3) % 4
            pltpu.make_async_copy(y_src.at[row_v.at[i_cur]], buf, gsem).wait()
            pltpu.async_copy(buf, z_sp.at[col_v.at[i_cur]], ssem, add=True)

            @pl.when(jj >= 1)
            def _drain_prev():
                pltpu.make_async_copy(
                    nbuf, z_sp.at[col_v.at[i_old]], ssem).wait()

            @pl.when(jj + 1 < NCHT)
            def _prefetch():
                pltpu.make_async_copy(
                    ei_hbm.at[0, sid, jj + 1], row_v.at[i_nxt], isem).wait()
                pltpu.make_async_copy(
                    ei_hbm.at[1, sid, jj + 1], col_v.at[i_nxt], isem).wait()
                pltpu.async_copy(y_src.at[row_v.at[i_nxt]], nbuf, gsem)

            @pl.when(jj + 3 < NCHT)
            def _stage_idx():
                pltpu.async_copy(ei_hbm.at[0, sid, jj + 3], row_v.at[i_old],
                                 isem)
                pltpu.async_copy(ei_hbm.at[1, sid, jj + 3], col_v.at[i_old],
                                 isem)

    pltpu.make_async_copy(bufs[(NCHT - 1) % 2],
                          z_sp.at[col_v.at[3]], ssem).wait()
    plsc.subcore_barrier()
    pltpu.sync_copy(z_sp.at[pl.ds(sid * RT, RT)],
                    z_out.at[cid, pl.ds(sid * RT, RT)])


_TB = 2000


def _lin_body(x_ref, w_ref, b_ref, degs_ref, yh_ref, dis_ref):
    xw = lax.dot_general(x_ref[...], w_ref[...], (((1,), (1,)), ((), ())),
                         preferred_element_type=jnp.float32)
    xw = xw + b_ref[...]
    deg = degs_ref[0, :, 0] + degs_ref[1, :, 0] + 1.0
    dis = lax.rsqrt(deg)[:, None]
    y = dis * xw
    yh_ref[0] = y[:, :DH]
    yh_ref[1] = y[:, DH:]
    dis_ref[...] = dis


def _lin_call(x, W, b2, deg_parts):
    return pl.pallas_call(
        _lin_body,
        grid=(N // _TB,),
        in_specs=[
            pl.BlockSpec((_TB, D), lambda i: (i, 0)),
            pl.BlockSpec((D, D), lambda i: (0, 0)),
            pl.BlockSpec((1, D), lambda i: (0, 0)),
            pl.BlockSpec((NC, _TB, 8), lambda i: (0, i, 0)),
        ],
        out_specs=[
            pl.BlockSpec((NC, _TB, DH), lambda i: (0, i, 0)),
            pl.BlockSpec((_TB, 1), lambda i: (i, 0)),
        ],
        out_shape=[
            jax.ShapeDtypeStruct((NC, N, DH), jnp.float32),
            jax.ShapeDtypeStruct((N, 1), jnp.float32),
        ],
    )(x, W, b2, deg_parts)


def _bn_body(zp_ref, yh_ref, dis_ref, g_ref, be_ref, o_ref):
    tl = zp_ref[0, :N, :] + yh_ref[0]
    tr = zp_ref[1, :N, :] + yh_ref[1]
    t = jnp.concatenate([tl, tr], axis=1) * dis_ref[...]
    m = jnp.mean(t, axis=0, keepdims=True)
    d = t - m
    v = jnp.mean(d * d, axis=0, keepdims=True)
    o_ref[...] = jnp.maximum(
        d * lax.rsqrt(v + BN_EPS) * g_ref[...] + be_ref[...], 0.0)


def _bn_call(z_parts, yh, dis, g2, be2):
    return pl.pallas_call(
        _bn_body,
        out_shape=jax.ShapeDtypeStruct((N, D), jnp.float32),
    )(z_parts, yh, dis, g2, be2)


def kernel(x, edge_index, W, b, bn_gamma, bn_beta):
    eic = edge_index.astype(jnp.int32).reshape(2, NS, NCHT, CW)
    zeros = jnp.zeros((RT, DH), jnp.float32)
    ones8 = jnp.ones((CW, DW), jnp.float32)
    zeros8 = jnp.zeros((RT, DW), jnp.float32)

    deg_parts = _deg_kernel(eic, ones8, zeros8)
    yh, dis = _lin_call(x, W, b.reshape(1, D), deg_parts)
    z_parts = _msg_kernel(yh, eic, zeros)
    out = _bn_call(z_parts, yh, dis,
                   bn_gamma.reshape(1, D), bn_beta.reshape(1, D))
    return out
